# baseline (speedup 1.0000x reference)
import functools

import jax
import jax.numpy as jnp
from jax import lax
from jax.experimental import pallas as pl
from jax.experimental.pallas import tpu as pltpu
from jax.experimental.pallas import tpu_sc as plsc

NC = 2
NS = 16
NW = NC * NS
LANES = 16
CH = 128
PB = 40
NBLK = 64
EPAD = NBLK * PB * CH
CPT = 4


def kernel(X, edge_index, edge_weight, W, b):
    N, D = X.shape
    E = edge_index.shape[1]
    DO = W.shape[1]
    pad = EPAD - E
    src_r = jnp.concatenate(
        [edge_index[0], jnp.zeros((pad,), jnp.int32)]).reshape(EPAD // CH, CH)
    dst_r = jnp.concatenate(
        [edge_index[1], jnp.zeros((pad,), jnp.int32)]).reshape(EPAD // CH, CH)
    w_p = jnp.concatenate([edge_weight, jnp.zeros((pad,), jnp.float32)])
    XT = jnp.transpose(X)
    fT = _sc_spmm_cols(XT, src_r, dst_r, w_p, N=N, D=D)
    out = _tc_linear(fT, W, b.reshape(1, DO), N=N, D=D, DO=DO, BM=1000)
    return out


def _sc_spmm_cols(XT, src_r, dst_r, w_p, *, N, D):
    IT = PB * CH // LANES

    mesh = plsc.VectorSubcoreMesh(core_axis_name="c", subcore_axis_name="s")

    @functools.partial(
        pl.kernel,
        out_type=jax.ShapeDtypeStruct((D, N), jnp.float32),
        mesh=mesh,
        compiler_params=pltpu.CompilerParams(use_tc_tiling_on_sc=False,
                                             needs_layout_passes=False),
        scratch_types=[
            pltpu.VMEM((CPT, N), jnp.float32),
            pltpu.VMEM((CPT, N), jnp.float32),
            pltpu.VMEM((PB, CH), jnp.int32),
            pltpu.VMEM((PB, CH), jnp.int32),
            pltpu.VMEM((PB * CH,), jnp.float32),
            pltpu.VMEM((PB, CH), jnp.int32),
            pltpu.VMEM((PB, CH), jnp.int32),
            pltpu.VMEM((PB * CH,), jnp.float32),
            pltpu.SemaphoreType.DMA,
            pltpu.SemaphoreType.DMA,
        ],
    )
    def spmm(xt_hbm, src_hbm, dst_hbm, w_hbm, ft_hbm,
             xt4, ft4, sA, dA, wA, sB, dB, wB, semA, semB):
        c = lax.axis_index("c")
        s = lax.axis_index("s")
        wid = s * NC + c
        col0 = wid * CPT

        pltpu.sync_copy(xt_hbm.at[pl.ds(col0, CPT)], xt4)
        zero = jnp.zeros((LANES,), jnp.float32)

        def zero_body(i, carry):
            for col in range(CPT):
                ft4[col, pl.ds(i * LANES, LANES)] = zero
            return carry

        lax.fori_loop(0, N // LANES, zero_body, 0)

        def stage(blk, sbuf, dbuf, wbuf, sem):
            pltpu.async_copy(src_hbm.at[pl.ds(blk * PB, PB)], sbuf, sem)
            pltpu.async_copy(dst_hbm.at[pl.ds(blk * PB, PB)], dbuf, sem)
            pltpu.async_copy(w_hbm.at[pl.ds(blk * PB * CH, PB * CH)],
                             wbuf, sem)

        def stage_wait(sbuf, dbuf, wbuf, sem):
            pltpu.make_async_copy(src_hbm.at[pl.ds(0, PB)], sbuf, sem).wait()
            pltpu.make_async_copy(dst_hbm.at[pl.ds(0, PB)], dbuf, sem).wait()
            pltpu.make_async_copy(w_hbm.at[pl.ds(0, PB * CH)],
                                  wbuf, sem).wait()

        def process(sbuf, dbuf, wbuf):
            def row_body(r, carry):
                for g in range(CH // LANES):
                    sl = pl.ds(g * LANES, LANES)
                    src16 = sbuf[r, sl]
                    dst16 = dbuf[r, sl]
                    w16 = wbuf[pl.ds(r * CH + g * LANES, LANES)]
                    for col in range(CPT):
                        rowi = jnp.full((LANES,), col, jnp.int32)
                        xv = plsc.load_gather(xt4, [rowi, src16])
                        plsc.addupdate_scatter(ft4, [rowi, dst16], xv * w16)
                return carry

            lax.fori_loop(0, PB, row_body, 0)

        stage(0, sA, dA, wA, semA)
        stage(1, sB, dB, wB, semB)

        def blk_body(q, carry):
            blk = q * 2
            stage_wait(sA, dA, wA, semA)
            process(sA, dA, wA)

            @pl.when(q < NBLK // 2 - 1)
            def _():
                stage(blk + 2, sA, dA, wA, semA)

            stage_wait(sB, dB, wB, semB)
            process(sB, dB, wB)

            @pl.when(q < NBLK // 2 - 1)
            def _():
                stage(blk + 3, sB, dB, wB, semB)
            return carry

        lax.fori_loop(0, NBLK // 2, blk_body, 0)

        pltpu.sync_copy(ft4, ft_hbm.at[pl.ds(col0, CPT)])

    return spmm(XT, src_r, dst_r, w_p)


def _tc_linear(fT, W, b2, *, N, D, DO, BM):
    def body(f_ref, w_ref, b_ref, o_ref):
        o_ref[...] = lax.dot_general(
            f_ref[...], w_ref[...],
            dimension_numbers=(((0,), (0,)), ((), ())),
            preferred_element_type=jnp.float32) + b_ref[...]

    return pl.pallas_call(
        body,
        in_specs=[
            pl.BlockSpec((D, N), lambda: (0, 0)),
            pl.BlockSpec((D, DO), lambda: (0, 0)),
            pl.BlockSpec((1, DO), lambda: (0, 0)),
        ],
        out_specs=pl.BlockSpec((N, DO), lambda: (0, 0)),
        out_shape=jax.ShapeDtypeStruct((N, DO), jnp.float32),
    )(fT, W, b2)


# speedup vs baseline: 2.1887x; 2.1887x over previous
import functools

import jax
import jax.numpy as jnp
from jax import lax
from jax.experimental import pallas as pl
from jax.experimental.pallas import tpu as pltpu
from jax.experimental.pallas import tpu_sc as plsc

NC = 2
NS = 16
NW = NC * NS
LANES = 16
CH = 128
PB = 40
NBLK = 64
EPAD = NBLK * PB * CH
CPT = 4


def kernel(X, edge_index, edge_weight, W, b):
    N, D = X.shape
    E = edge_index.shape[1]
    DO = W.shape[1]
    pad = EPAD - E
    src_r = jnp.concatenate(
        [edge_index[0], jnp.zeros((pad,), jnp.int32)]).reshape(EPAD // CH, CH)
    dst_r = jnp.concatenate(
        [edge_index[1], jnp.zeros((pad,), jnp.int32)]).reshape(EPAD // CH, CH)
    w_p = jnp.concatenate([edge_weight, jnp.zeros((pad,), jnp.float32)])
    XT = jnp.transpose(X)
    fT = _sc_spmm_cols(XT, src_r, dst_r, w_p, N=N, D=D)
    out = _tc_linear(fT, W, b.reshape(1, DO), N=N, D=D, DO=DO, BM=1000)
    return out


def _sc_spmm_cols(XT, src_r, dst_r, w_p, *, N, D):
    IT = PB * CH // LANES

    mesh = plsc.VectorSubcoreMesh(core_axis_name="c", subcore_axis_name="s")

    @functools.partial(
        pl.kernel,
        out_type=jax.ShapeDtypeStruct((D, N), jnp.float32),
        mesh=mesh,
        compiler_params=pltpu.CompilerParams(use_tc_tiling_on_sc=False,
                                             needs_layout_passes=False),
        scratch_types=[
            pltpu.VMEM((CPT, N), jnp.float32),
            pltpu.VMEM((CPT, N), jnp.float32),
            pltpu.VMEM((PB, CH), jnp.int32),
            pltpu.VMEM((PB, CH), jnp.int32),
            pltpu.VMEM((PB * CH,), jnp.float32),
            pltpu.VMEM((PB, CH), jnp.int32),
            pltpu.VMEM((PB, CH), jnp.int32),
            pltpu.VMEM((PB * CH,), jnp.float32),
            pltpu.SemaphoreType.DMA,
            pltpu.SemaphoreType.DMA,
        ],
    )
    def spmm(xt_hbm, src_hbm, dst_hbm, w_hbm, ft_hbm,
             xt4, ft4, sA, dA, wA, sB, dB, wB, semA, semB):
        c = lax.axis_index("c")
        s = lax.axis_index("s")
        wid = s * NC + c
        col0 = wid * CPT

        pltpu.sync_copy(xt_hbm.at[pl.ds(col0, CPT)], xt4)
        zero = jnp.zeros((LANES,), jnp.float32)

        def zero_body(i, carry):
            for col in range(CPT):
                ft4[col, pl.ds(i * LANES, LANES)] = zero
            return carry

        lax.fori_loop(0, N // LANES, zero_body, 0)

        def stage(blk, sbuf, dbuf, wbuf, sem):
            pltpu.async_copy(src_hbm.at[pl.ds(blk * PB, PB)], sbuf, sem)
            pltpu.async_copy(dst_hbm.at[pl.ds(blk * PB, PB)], dbuf, sem)
            pltpu.async_copy(w_hbm.at[pl.ds(blk * PB * CH, PB * CH)],
                             wbuf, sem)

        def stage_wait(sbuf, dbuf, wbuf, sem):
            pltpu.make_async_copy(src_hbm.at[pl.ds(0, PB)], sbuf, sem).wait()
            pltpu.make_async_copy(dst_hbm.at[pl.ds(0, PB)], dbuf, sem).wait()
            pltpu.make_async_copy(w_hbm.at[pl.ds(0, PB * CH)],
                                  wbuf, sem).wait()

        def process(sbuf, dbuf, wbuf):
            @plsc.parallel_loop(0, PB, 1, unroll=2)
            def _row(r):
                for g in range(CH // LANES):
                    sl = pl.ds(g * LANES, LANES)
                    src16 = sbuf[r, sl]
                    dst16 = dbuf[r, sl]
                    w16 = wbuf[pl.ds(r * CH + g * LANES, LANES)]
                    for col in range(CPT):
                        rowi = jnp.full((LANES,), col, jnp.int32)
                        xv = plsc.load_gather(xt4, [rowi, src16])
                        plsc.addupdate_scatter(ft4, [rowi, dst16], xv * w16)

        stage(0, sA, dA, wA, semA)
        stage(1, sB, dB, wB, semB)

        def blk_body(q, carry):
            blk = q * 2
            stage_wait(sA, dA, wA, semA)
            process(sA, dA, wA)

            @pl.when(q < NBLK // 2 - 1)
            def _():
                stage(blk + 2, sA, dA, wA, semA)

            stage_wait(sB, dB, wB, semB)
            process(sB, dB, wB)

            @pl.when(q < NBLK // 2 - 1)
            def _():
                stage(blk + 3, sB, dB, wB, semB)
            return carry

        lax.fori_loop(0, NBLK // 2, blk_body, 0)

        pltpu.sync_copy(ft4, ft_hbm.at[pl.ds(col0, CPT)])

    return spmm(XT, src_r, dst_r, w_p)


def _tc_linear(fT, W, b2, *, N, D, DO, BM):
    def body(f_ref, w_ref, b_ref, o_ref):
        o_ref[...] = lax.dot_general(
            f_ref[...], w_ref[...],
            dimension_numbers=(((0,), (0,)), ((), ())),
            preferred_element_type=jnp.float32) + b_ref[...]

    return pl.pallas_call(
        body,
        in_specs=[
            pl.BlockSpec((D, N), lambda: (0, 0)),
            pl.BlockSpec((D, DO), lambda: (0, 0)),
            pl.BlockSpec((1, DO), lambda: (0, 0)),
        ],
        out_specs=pl.BlockSpec((N, DO), lambda: (0, 0)),
        out_shape=jax.ShapeDtypeStruct((N, DO), jnp.float32),
    )(fT, W, b2)


# speedup vs baseline: 2.3126x; 1.0566x over previous
import functools

import jax
import jax.numpy as jnp
from jax import lax
from jax.experimental import pallas as pl
from jax.experimental.pallas import tpu as pltpu
from jax.experimental.pallas import tpu_sc as plsc

NC = 2
NS = 16
NW = NC * NS
LANES = 16
CH = 128
PB = 40
NBLK = 64
EPAD = NBLK * PB * CH
CPT = 4


def kernel(X, edge_index, edge_weight, W, b):
    N, D = X.shape
    E = edge_index.shape[1]
    DO = W.shape[1]
    pad = EPAD - E
    src_r = jnp.concatenate(
        [edge_index[0], jnp.zeros((pad,), jnp.int32)]).reshape(EPAD // CH, CH)
    dst_r = jnp.concatenate(
        [edge_index[1], jnp.zeros((pad,), jnp.int32)]).reshape(EPAD // CH, CH)
    w_p = jnp.concatenate([edge_weight, jnp.zeros((pad,), jnp.float32)])
    XT = jnp.transpose(X)
    fT = _sc_spmm_cols(XT, src_r, dst_r, w_p, N=N, D=D)
    out = _tc_linear(fT, W, b.reshape(1, DO), N=N, D=D, DO=DO, BM=1000)
    return out


def _sc_spmm_cols(XT, src_r, dst_r, w_p, *, N, D):
    IT = PB * CH // LANES

    mesh = plsc.VectorSubcoreMesh(core_axis_name="c", subcore_axis_name="s")

    @functools.partial(
        pl.kernel,
        out_type=jax.ShapeDtypeStruct((D, N), jnp.float32),
        mesh=mesh,
        compiler_params=pltpu.CompilerParams(use_tc_tiling_on_sc=False,
                                             needs_layout_passes=False),
        scratch_types=[
            pltpu.VMEM((CPT, N), jnp.float32),
            pltpu.VMEM((CPT, N), jnp.float32),
            pltpu.VMEM((PB, CH), jnp.int32),
            pltpu.VMEM((PB, CH), jnp.int32),
            pltpu.VMEM((PB * CH,), jnp.float32),
            pltpu.VMEM((PB, CH), jnp.int32),
            pltpu.VMEM((PB, CH), jnp.int32),
            pltpu.VMEM((PB * CH,), jnp.float32),
            pltpu.SemaphoreType.DMA,
            pltpu.SemaphoreType.DMA,
        ],
    )
    def spmm(xt_hbm, src_hbm, dst_hbm, w_hbm, ft_hbm,
             xt4, ft4, sA, dA, wA, sB, dB, wB, semA, semB):
        c = lax.axis_index("c")
        s = lax.axis_index("s")
        wid = s * NC + c
        col0 = wid * CPT

        pltpu.sync_copy(xt_hbm.at[pl.ds(col0, CPT)], xt4)
        zero = jnp.zeros((LANES,), jnp.float32)

        def zero_body(i, carry):
            for col in range(CPT):
                ft4[col, pl.ds(i * LANES, LANES)] = zero
            return carry

        lax.fori_loop(0, N // LANES, zero_body, 0)

        def stage(blk, sbuf, dbuf, wbuf, sem):
            pltpu.async_copy(src_hbm.at[pl.ds(blk * PB, PB)], sbuf, sem)
            pltpu.async_copy(dst_hbm.at[pl.ds(blk * PB, PB)], dbuf, sem)
            pltpu.async_copy(w_hbm.at[pl.ds(blk * PB * CH, PB * CH)],
                             wbuf, sem)

        def stage_wait(sbuf, dbuf, wbuf, sem):
            pltpu.make_async_copy(src_hbm.at[pl.ds(0, PB)], sbuf, sem).wait()
            pltpu.make_async_copy(dst_hbm.at[pl.ds(0, PB)], dbuf, sem).wait()
            pltpu.make_async_copy(w_hbm.at[pl.ds(0, PB * CH)],
                                  wbuf, sem).wait()

        def process(sbuf, dbuf, wbuf):
            @plsc.parallel_loop(0, PB, 1, unroll=4)
            def _row(r):
                for g in range(CH // LANES):
                    sl = pl.ds(g * LANES, LANES)
                    src16 = sbuf[r, sl]
                    dst16 = dbuf[r, sl]
                    w16 = wbuf[pl.ds(r * CH + g * LANES, LANES)]
                    for col in range(CPT):
                        rowi = jnp.full((LANES,), col, jnp.int32)
                        xv = plsc.load_gather(xt4, [rowi, src16])
                        plsc.addupdate_scatter(ft4, [rowi, dst16], xv * w16)

        stage(0, sA, dA, wA, semA)
        stage(1, sB, dB, wB, semB)

        def blk_body(q, carry):
            blk = q * 2
            stage_wait(sA, dA, wA, semA)
            process(sA, dA, wA)

            @pl.when(q < NBLK // 2 - 1)
            def _():
                stage(blk + 2, sA, dA, wA, semA)

            stage_wait(sB, dB, wB, semB)
            process(sB, dB, wB)

            @pl.when(q < NBLK // 2 - 1)
            def _():
                stage(blk + 3, sB, dB, wB, semB)
            return carry

        lax.fori_loop(0, NBLK // 2, blk_body, 0)

        pltpu.sync_copy(ft4, ft_hbm.at[pl.ds(col0, CPT)])

    return spmm(XT, src_r, dst_r, w_p)


def _tc_linear(fT, W, b2, *, N, D, DO, BM):
    def body(f_ref, w_ref, b_ref, o_ref):
        o_ref[...] = lax.dot_general(
            f_ref[...], w_ref[...],
            dimension_numbers=(((0,), (0,)), ((), ())),
            preferred_element_type=jnp.float32) + b_ref[...]

    return pl.pallas_call(
        body,
        in_specs=[
            pl.BlockSpec((D, N), lambda: (0, 0)),
            pl.BlockSpec((D, DO), lambda: (0, 0)),
            pl.BlockSpec((1, DO), lambda: (0, 0)),
        ],
        out_specs=pl.BlockSpec((N, DO), lambda: (0, 0)),
        out_shape=jax.ShapeDtypeStruct((N, DO), jnp.float32),
    )(fT, W, b2)
